# baseline (speedup 1.0000x reference)
import functools

import jax
import jax.numpy as jnp
from jax import lax
from jax.experimental import pallas as pl
from jax.experimental.pallas import tpu as pltpu
from jax.experimental.pallas import tpu_sc as plsc

N_NODES = 10000
N_EDGES = 160000
IN_DIM = 256
HID_DIM = 512

HALF = IN_DIM // 2
CW = 16
NC = 2
NS = 16
EDGES_PER_TILE = N_EDGES // NS
CHUNK = 125
NCHUNK = EDGES_PER_TILE // CHUNK
ROWS_PER_TILE = N_NODES // NS


def _sc_scatter(emb2, ed, ones_h, zf, zc, feat, cnt,
                idx_a, idx_b, rows_a, rows_b, ones_v, shf, shc,
                sem_ga, sem_gb, sem_sa, sem_sb, sem_oa, sem_ob):
    c = lax.axis_index("c")
    s = lax.axis_index("s")
    pltpu.sync_copy(zf, shf.at[pl.ds(s * ROWS_PER_TILE, ROWS_PER_TILE)])
    pltpu.sync_copy(zc, shc.at[pl.ds(s * ROWS_PER_TILE, ROWS_PER_TILE)])
    pltpu.sync_copy(ones_h, ones_v)
    plsc.subcore_barrier()

    NJJ = NCHUNK // 2
    first_jj = jnp.where(c == 1, NJJ // 2, 0)

    def ones_scatter(jj, j, idx_v, sem_o):
        @pl.when(jnp.logical_xor(j < NCHUNK // 2, c == 1))
        def _():
            @pl.when(jj > first_jj)
            def _():
                pltpu.make_async_copy(ones_h, ones_v, sem_o).wait()
            pltpu.async_copy(ones_v, shc.at[idx_v.at[0]], sem_o, add=True)

    pltpu.sync_copy(ed.at[c, s, 0], idx_a)
    pltpu.async_copy(emb2.at[idx_a.at[1]], rows_a, sem_ga)

    def body(jj, carry):
        a = 2 * jj
        b = a + 1
        @pl.when(jj > 0)
        def _():
            pltpu.make_async_copy(emb2.at[pl.ds(0, CHUNK)], rows_b, sem_sb).wait()
        pltpu.sync_copy(ed.at[c, s, b], idx_b)
        pltpu.async_copy(emb2.at[idx_b.at[1]], rows_b, sem_gb)
        pltpu.make_async_copy(emb2.at[pl.ds(0, CHUNK)], rows_a, sem_ga).wait()
        pltpu.async_copy(rows_a, shf.at[idx_a.at[0]], sem_sa, add=True)
        ones_scatter(jj, a, idx_a, sem_oa)
        pltpu.make_async_copy(emb2.at[pl.ds(0, CHUNK)], rows_a, sem_sa).wait()

        @pl.when(jj + 1 < NJJ)
        def _():
            pltpu.sync_copy(ed.at[c, s, a + 2], idx_a)
            pltpu.async_copy(emb2.at[idx_a.at[1]], rows_a, sem_ga)
        pltpu.make_async_copy(emb2.at[pl.ds(0, CHUNK)], rows_b, sem_gb).wait()
        pltpu.async_copy(rows_b, shf.at[idx_b.at[0]], sem_sb, add=True)
        ones_scatter(jj, b, idx_b, sem_ob)
        return carry

    lax.fori_loop(0, NJJ, body, 0)
    pltpu.make_async_copy(emb2.at[pl.ds(0, CHUNK)], rows_b, sem_sb).wait()
    pltpu.make_async_copy(ones_h, ones_v, sem_oa).wait()
    pltpu.make_async_copy(ones_h, ones_v, sem_ob).wait()
    plsc.subcore_barrier()
    pltpu.sync_copy(
        shf.at[pl.ds(s * ROWS_PER_TILE, ROWS_PER_TILE)],
        feat.at[c, pl.ds(s * ROWS_PER_TILE, ROWS_PER_TILE)],
    )
    pltpu.sync_copy(
        shc.at[pl.ds(s * ROWS_PER_TILE, ROWS_PER_TILE)],
        cnt.at[c, pl.ds(s * ROWS_PER_TILE, ROWS_PER_TILE)],
    )


_sc_scatter_call = functools.partial(
    pl.kernel,
    out_type=(
        jax.ShapeDtypeStruct((NC, N_NODES, HALF), jnp.float32),
        jax.ShapeDtypeStruct((NC, N_NODES, CW), jnp.float32),
    ),
    mesh=plsc.VectorSubcoreMesh(core_axis_name="c", subcore_axis_name="s"),
    scratch_types=[
        pltpu.VMEM((2, CHUNK), jnp.int32),
        pltpu.VMEM((2, CHUNK), jnp.int32),
        pltpu.VMEM((CHUNK, HALF), jnp.float32),
        pltpu.VMEM((CHUNK, HALF), jnp.float32),
        pltpu.VMEM((CHUNK, CW), jnp.float32),
        pltpu.VMEM_SHARED((N_NODES, HALF), jnp.float32),
        pltpu.VMEM_SHARED((N_NODES, CW), jnp.float32),
        pltpu.SemaphoreType.DMA,
        pltpu.SemaphoreType.DMA,
        pltpu.SemaphoreType.DMA,
        pltpu.SemaphoreType.DMA,
        pltpu.SemaphoreType.DMA,
        pltpu.SemaphoreType.DMA,
    ],
    compiler_params=pltpu.CompilerParams(use_tc_tiling_on_sc=False),
)(_sc_scatter)


def _tc_body(x_ref, f_ref, c_ref, ws_ref, wn0_ref, wn1_ref, b_ref, o_ref):
    x = x_ref[...]
    fb = f_ref[...]
    cb = c_ref[...]
    cnt = cb[0, :, :1] + cb[1, :, :1]
    mask = cnt > 0.0
    safe = jnp.where(mask, cnt, 1.0)
    m0 = jnp.where(mask, fb[0] / safe, 0.0)
    m1 = jnp.where(mask, fb[1] / safe, 0.0)
    acc = jnp.dot(x, ws_ref[...], preferred_element_type=jnp.float32)
    acc += jnp.dot(m0, wn0_ref[...], preferred_element_type=jnp.float32)
    acc += jnp.dot(m1, wn1_ref[...], preferred_element_type=jnp.float32)
    acc += b_ref[...]
    acc = jnp.maximum(acc, 0.0)
    nrm = jnp.sqrt(jnp.sum(acc * acc, axis=1, keepdims=True)) + 1e-9
    o_ref[...] = acc / nrm


def _tc_call(x, feat, cnt, ws, wn0, wn1, b):
    R = 1000
    grid = (N_NODES // R,)
    return pl.pallas_call(
        _tc_body,
        grid=grid,
        in_specs=[
            pl.BlockSpec((R, IN_DIM), lambda i: (i, 0)),
            pl.BlockSpec((NC, R, HALF), lambda i: (0, i, 0)),
            pl.BlockSpec((NC, R, CW), lambda i: (0, i, 0)),
            pl.BlockSpec((IN_DIM, HID_DIM), lambda i: (0, 0)),
            pl.BlockSpec((HALF, HID_DIM), lambda i: (0, 0)),
            pl.BlockSpec((HALF, HID_DIM), lambda i: (0, 0)),
            pl.BlockSpec((1, HID_DIM), lambda i: (0, 0)),
        ],
        out_specs=pl.BlockSpec((R, HID_DIM), lambda i: (i, 0)),
        out_shape=jax.ShapeDtypeStruct((N_NODES, HID_DIM), jnp.float32),
    )(x, feat, cnt, ws, wn0, wn1, b)


@jax.jit
def kernel(item_emb, edges, w_self_W, w_self_b, w_neigh_W, w_neigh_b):
    f32 = jnp.float32
    src = edges[:, 0].astype(jnp.int32)
    dst = edges[:, 1].astype(jnp.int32)
    emb2 = item_emb.reshape(2 * N_NODES, HALF)
    srcr = src.reshape(NS, NCHUNK, 1, CHUNK)
    d2 = 2 * dst
    ed = jnp.stack([
        jnp.concatenate([srcr, d2.reshape(NS, NCHUNK, 1, CHUNK)], axis=2),
        jnp.concatenate([srcr, (d2 + 1).reshape(NS, NCHUNK, 1, CHUNK)], axis=2),
    ])
    ones_h = jnp.ones((CHUNK, CW), f32)
    zf = jnp.zeros((ROWS_PER_TILE, HALF), f32)
    zc = jnp.zeros((ROWS_PER_TILE, CW), f32)

    feat, cnt = _sc_scatter_call(emb2, ed, ones_h, zf, zc)

    bias = (w_self_b + w_neigh_b).reshape(1, HID_DIM)
    return _tc_call(item_emb, feat, cnt, w_self_W,
                    w_neigh_W[:HALF], w_neigh_W[HALF:], bias)
